# baseline (speedup 1.0000x reference)
import functools

import jax
import jax.numpy as jnp
from jax import lax
from jax.experimental import pallas as pl
from jax.experimental.pallas import tpu as pltpu
from jax.experimental.pallas import tpu_sc as plsc

_D = 1024
_E = 8
_F = 1024
_N = 2048
_L = 16
_NW = 32
_TPW = _N // _NW
_NCHUNK = _TPW // _L
_BLK = 512
_NBLK = 15
_S = _NBLK * _BLK
_BEXP_PAD = 48


def _gelu_tanh(h):
    c = 0.7978845608028654
    return 0.5 * h * (1.0 + jnp.tanh(c * (h + 0.044715 * h * h * h)))


def _router_kernel(x_ref, wr_ref, e0_ref, e1_ref, p0_ref, p1_ref,
                   run_ref, bexp_ref):
    xf = x_ref[...]
    logits = lax.dot_general(wr_ref[...], xf, (((1,), (1,)), ((), ())),
                             preferred_element_type=jnp.float32)
    m = jnp.max(logits, axis=0, keepdims=True)
    ex = jnp.exp(logits - m)
    sm = ex / jnp.sum(ex, axis=0, keepdims=True)

    e0 = jnp.zeros((1, _N), jnp.int32)
    e1 = jnp.zeros((1, _N), jnp.int32)
    p0 = jnp.zeros((1, _N), jnp.float32)
    p1 = jnp.zeros((1, _N), jnp.float32)
    chosen_rows = []
    for j in range(_E):
        sj = sm[j:j + 1, :]
        rank = jnp.sum((sm > sj).astype(jnp.int32), axis=0, keepdims=True)
        if j > 0:
            rank = rank + jnp.sum((sm[:j, :] == sj).astype(jnp.int32),
                                  axis=0, keepdims=True)
        is0 = rank == 0
        is1 = rank == 1
        e0 = jnp.where(is0, j, e0)
        p0 = jnp.where(is0, sj, p0)
        e1 = jnp.where(is1, j, e1)
        p1 = jnp.where(is1, sj, p1)
        chosen_rows.append((rank < 2).astype(jnp.float32))
    chosen = jnp.concatenate(chosen_rows, axis=0)

    row = lax.broadcasted_iota(jnp.int32, (_NW, _N), 1)
    wid = lax.broadcasted_iota(jnp.int32, (_NW, _N), 0)
    seg = (row // _TPW == wid).astype(jnp.float32)
    ws = lax.dot_general(seg, chosen, (((1,), (1,)), ((), ())),
                         preferred_element_type=jnp.float32)
    a = lax.broadcasted_iota(jnp.int32, (_NW, _NW), 0)
    b = lax.broadcasted_iota(jnp.int32, (_NW, _NW), 1)
    ltri = (b < a).astype(jnp.float32)
    pre = lax.dot_general(ltri, ws, (((1,), (0,)), ((), ())),
                          preferred_element_type=jnp.float32)
    total = jnp.sum(ws, axis=0, keepdims=True)
    blk = (total.astype(jnp.int32) + (_BLK - 1)) // _BLK
    ea = lax.broadcasted_iota(jnp.int32, (_E, _E), 0)
    eb = lax.broadcasted_iota(jnp.int32, (_E, _E), 1)
    ltri_e = (ea < eb).astype(jnp.float32)
    base_blk = lax.dot_general(blk.astype(jnp.float32), ltri_e,
                               (((1,), (0,)), ((), ())),
                               preferred_element_type=jnp.float32)
    base_blk = base_blk.astype(jnp.int32)
    off = base_blk * _BLK

    run0 = (off.astype(jnp.float32) + pre).astype(jnp.int32)
    run_ref[:, 0:_E] = run0
    run_ref[:, _E:_L] = jnp.zeros((_NW, _L - _E), jnp.int32)

    biota = lax.broadcasted_iota(jnp.int32, (1, _BEXP_PAD), 1)
    bexp = jnp.zeros((1, _BEXP_PAD), jnp.int32)
    for e in range(_E):
        be = base_blk[:, e:e + 1]
        ne = blk[:, e:e + 1]
        mask = (biota >= be) & (biota < be + ne)
        bexp = jnp.where(mask, e, bexp)
    bexp_ref[...] = bexp

    e0_ref[...] = e0
    e1_ref[...] = e1
    p0_ref[...] = p0
    p1_ref[...] = p1


def _router(x2, Wr):
    return pl.pallas_call(
        _router_kernel,
        out_shape=(
            jax.ShapeDtypeStruct((1, _N), jnp.int32),
            jax.ShapeDtypeStruct((1, _N), jnp.int32),
            jax.ShapeDtypeStruct((1, _N), jnp.float32),
            jax.ShapeDtypeStruct((1, _N), jnp.float32),
            jax.ShapeDtypeStruct((_NW, _L), jnp.int32),
            jax.ShapeDtypeStruct((1, _BEXP_PAD), jnp.int32),
        ),
    )(x2, Wr)


def _dispatch_body(x_hbm, e0_hbm, e1_hbm, run_hbm, xg_hbm, s0_hbm, s1_hbm,
                   e0_v, e1_v, s0_v, s1_v, run_v, xrows):
    w = lax.axis_index("s") * _NC_CORES + lax.axis_index("c")
    base = w * _TPW
    pltpu.sync_copy(e0_hbm.at[pl.ds(base, _TPW)], e0_v)
    pltpu.sync_copy(e1_hbm.at[pl.ds(base, _TPW)], e1_v)
    pltpu.sync_copy(run_hbm.at[w], run_v)
    run = run_v[...]
    ivec = lax.iota(jnp.int32, _L)
    for ci in range(_NCHUNK):
        tok0 = base + ci * _L
        pltpu.sync_copy(x_hbm.at[pl.ds(tok0, _L)], xrows)
        for which in range(2):
            ev = e0_v if which == 0 else e1_v
            c = ev[pl.ds(ci * _L, _L)]
            slot = jnp.zeros((_L,), jnp.int32)
            for e in range(_E):
                msk = c == e
                pref = plsc.cumsum(msk.astype(jnp.int32))
                base_e = jnp.sum(jnp.where(ivec == e, run, 0))
                slot = jnp.where(msk, base_e + pref - 1, slot)
                cnt = plsc.all_reduce_population_count(msk)
                run = jnp.where(ivec == e, run + cnt, run)
            pltpu.sync_copy(xrows, xg_hbm.at[slot])
            if which == 0:
                s0_v[pl.ds(ci * _L, _L)] = slot
            else:
                s1_v[pl.ds(ci * _L, _L)] = slot
    pltpu.sync_copy(s0_v, s0_hbm.at[pl.ds(base, _TPW)])
    pltpu.sync_copy(s1_v, s1_hbm.at[pl.ds(base, _TPW)])


_NC_CORES = 2


@functools.cache
def _dispatch_call():
    mesh = plsc.VectorSubcoreMesh(core_axis_name="c", subcore_axis_name="s")
    return functools.partial(
        pl.kernel,
        out_type=(
            jax.ShapeDtypeStruct((_S, _D), jnp.float32),
            jax.ShapeDtypeStruct((_N,), jnp.int32),
            jax.ShapeDtypeStruct((_N,), jnp.int32),
        ),
        mesh=mesh,
        compiler_params=pltpu.CompilerParams(needs_layout_passes=False),
        scratch_types=[
            pltpu.VMEM((_TPW,), jnp.int32),
            pltpu.VMEM((_TPW,), jnp.int32),
            pltpu.VMEM((_TPW,), jnp.int32),
            pltpu.VMEM((_TPW,), jnp.int32),
            pltpu.VMEM((_L,), jnp.int32),
            pltpu.VMEM((_L, _D), jnp.float32),
        ],
    )(_dispatch_body)


def _wcast_kernel(w1_ref, w2_ref, w1b_ref, w2b_ref):
    w1b_ref[...] = w1_ref[...].astype(jnp.bfloat16)
    w2b_ref[...] = w2_ref[...].astype(jnp.bfloat16)


def _wcast(W1, W2):
    return pl.pallas_call(
        _wcast_kernel,
        grid=(_E,),
        in_specs=[
            pl.BlockSpec((1, _F, _D), lambda e: (e, 0, 0)),
            pl.BlockSpec((1, _D, _F), lambda e: (e, 0, 0)),
        ],
        out_specs=(
            pl.BlockSpec((1, _F, _D), lambda e: (e, 0, 0)),
            pl.BlockSpec((1, _D, _F), lambda e: (e, 0, 0)),
        ),
        out_shape=(
            jax.ShapeDtypeStruct((_E, _F, _D), jnp.bfloat16),
            jax.ShapeDtypeStruct((_E, _D, _F), jnp.bfloat16),
        ),
        compiler_params=pltpu.CompilerParams(
            dimension_semantics=("arbitrary",)),
    )(W1, W2)


def _ffn_kernel(bexp_ref, xg_ref, w1_ref, w2_ref, y_ref):
    h = lax.dot_general(xg_ref[...], w1_ref[0], (((1,), (1,)), ((), ())),
                        preferred_element_type=jnp.float32)
    h = _gelu_tanh(h)
    y = lax.dot_general(h, w2_ref[0], (((1,), (1,)), ((), ())),
                        preferred_element_type=jnp.float32)
    y_ref[...] = y.astype(jnp.bfloat16)


def _ffn(bexp, xg, W1b, W2b):
    grid_spec = pltpu.PrefetchScalarGridSpec(
        num_scalar_prefetch=1,
        grid=(_NBLK,),
        in_specs=[
            pl.BlockSpec((_BLK, _D), lambda b, be: (b, 0)),
            pl.BlockSpec((1, _F, _D), lambda b, be: (be[b], 0, 0)),
            pl.BlockSpec((1, _D, _F), lambda b, be: (be[b], 0, 0)),
        ],
        out_specs=pl.BlockSpec((_BLK, _D), lambda b, be: (b, 0)),
    )
    return pl.pallas_call(
        _ffn_kernel,
        grid_spec=grid_spec,
        out_shape=jax.ShapeDtypeStruct((_S, _D), jnp.bfloat16),
        compiler_params=pltpu.CompilerParams(
            dimension_semantics=("arbitrary",)),
    )(bexp, xg, W1b, W2b)


def _combine_body(y_hbm, s0_hbm, s1_hbm, p0_hbm, p1_hbm, out_hbm,
                  s0a_v, s1a_v, p0a_v, p1a_v, y0_v, y1_v, out_v,
                  sem_idx, sem_g0, sem_g1, sem_out):
    w = lax.axis_index("s") * _NC_CORES + lax.axis_index("c")
    base = w * _TPW
    d0 = pltpu.async_copy(s0_hbm.at[pl.ds(base, _TPW)], s0a_v, sem_idx)
    d1 = pltpu.async_copy(s1_hbm.at[pl.ds(base, _TPW)], s1a_v, sem_idx)
    d2 = pltpu.async_copy(p0_hbm.at[pl.ds(base, _TPW)], p0a_v, sem_idx)
    d3 = pltpu.async_copy(p1_hbm.at[pl.ds(base, _TPW)], p1a_v, sem_idx)
    d0.wait(); d1.wait(); d2.wait(); d3.wait()

    ivec = lax.iota(jnp.int32, _L)
    zf = jnp.zeros((_L,), jnp.float32)

    def issue_gathers(ci, buf):
        idx0 = s0a_v[pl.ds(ci * _L, _L)]
        idx1 = s1a_v[pl.ds(ci * _L, _L)]
        g0 = pltpu.async_copy(y_hbm.at[idx0], y0_v.at[buf], sem_g0)
        g1 = pltpu.async_copy(y_hbm.at[idx1], y1_v.at[buf], sem_g1)
        return g0, g1

    pend = issue_gathers(0, 0)
    wr_pend = None
    for ci in range(_NCHUNK):
        buf = ci % 2
        pend[0].wait()
        pend[1].wait()
        if ci + 1 < _NCHUNK:
            pend = issue_gathers(ci + 1, (ci + 1) % 2)
        p0vec = p0a_v[pl.ds(ci * _L, _L)]
        p1vec = p1a_v[pl.ds(ci * _L, _L)]

        def tbody(t, _):
            p0s = zf + jnp.sum(jnp.where(ivec == t, p0vec, zf))
            p1s = zf + jnp.sum(jnp.where(ivec == t, p1vec, zf))
            p0b = plsc.pack(p0s, p0s, format=plsc.PackFormat.INTERLEAVED)
            p1b = plsc.pack(p1s, p1s, format=plsc.PackFormat.INTERLEAVED)
            for j in range(_D // (2 * _L)):
                y0c = plsc.bitcast(y0_v[buf, t, pl.ds(j * _L, _L)], jnp.bfloat16)
                y1c = plsc.bitcast(y1_v[buf, t, pl.ds(j * _L, _L)], jnp.bfloat16)
                r = p0b * y0c + p1b * y1c
                out_v[buf, t, pl.ds(j * _L, _L)] = plsc.bitcast(r, jnp.int32)
            return 0

        lax.fori_loop(0, _L, tbody, 0)
        if wr_pend is not None:
            wr_pend.wait()
        wr_pend = pltpu.async_copy(
            out_v.at[buf], out_hbm.at[pl.ds(base + ci * _L, _L)], sem_out)
    wr_pend.wait()


@functools.cache
def _combine_call():
    mesh = plsc.VectorSubcoreMesh(core_axis_name="c", subcore_axis_name="s")
    return functools.partial(
        pl.kernel,
        out_type=jax.ShapeDtypeStruct((_N, _D // 2), jnp.int32),
        mesh=mesh,
        compiler_params=pltpu.CompilerParams(needs_layout_passes=False),
        scratch_types=[
            pltpu.VMEM((_TPW,), jnp.int32),
            pltpu.VMEM((_TPW,), jnp.int32),
            pltpu.VMEM((_TPW,), jnp.float32),
            pltpu.VMEM((_TPW,), jnp.float32),
            pltpu.VMEM((2, _L, _D // 2), jnp.int32),
            pltpu.VMEM((2, _L, _D // 2), jnp.int32),
            pltpu.VMEM((2, _L, _D // 2), jnp.int32),
            pltpu.SemaphoreType.DMA,
            pltpu.SemaphoreType.DMA,
            pltpu.SemaphoreType.DMA,
            pltpu.SemaphoreType.DMA,
        ],
    )(_combine_body)


@jax.jit
def kernel(x, Wr, W1, W2):
    Bb, Ll, Dd = x.shape
    x2 = x.reshape(_N, _D)
    e0, e1, p0, p1, run0, bexp = _router(x2, Wr)
    e0 = e0.reshape(_N)
    e1 = e1.reshape(_N)
    p0 = p0.reshape(_N)
    p1 = p1.reshape(_N)
    bexp = bexp.reshape(_BEXP_PAD)
    xg, s0, s1 = _dispatch_call()(x2, e0, e1, run0)
    y = _ffn(bexp, xg, W1, W2)
    y32 = lax.bitcast_convert_type(y.reshape(_S, _D // 2, 2), jnp.int32)
    out32 = _combine_call()(y32, s0, s1, p0, p1)
    out = lax.bitcast_convert_type(out32, jnp.bfloat16).reshape(_N, _D)
    return out.astype(jnp.float32).reshape(Bb, Ll, Dd)


# speedup vs baseline: 3.0729x; 3.0729x over previous
import functools

import jax
import jax.numpy as jnp
from jax import lax
from jax.experimental import pallas as pl
from jax.experimental.pallas import tpu as pltpu
from jax.experimental.pallas import tpu_sc as plsc

_D = 1024
_E = 8
_F = 1024
_N = 2048
_L = 16
_NW = 32
_TPW = _N // _NW
_NCHUNK = _TPW // _L
_BLK = 512
_NBLK = 15
_S = _NBLK * _BLK
_BEXP_PAD = 48


def _gelu_tanh(h):
    c = 0.7978845608028654
    return 0.5 * h * (1.0 + jnp.tanh(c * (h + 0.044715 * h * h * h)))


def _router_kernel(x_ref, wr_ref, e0_ref, e1_ref, p0_ref, p1_ref,
                   run_ref, bexp_ref):
    xf = x_ref[...]
    logits = lax.dot_general(wr_ref[...], xf, (((1,), (1,)), ((), ())),
                             preferred_element_type=jnp.float32)
    m = jnp.max(logits, axis=0, keepdims=True)
    ex = jnp.exp(logits - m)
    sm = ex / jnp.sum(ex, axis=0, keepdims=True)

    e0 = jnp.zeros((1, _N), jnp.int32)
    e1 = jnp.zeros((1, _N), jnp.int32)
    p0 = jnp.zeros((1, _N), jnp.float32)
    p1 = jnp.zeros((1, _N), jnp.float32)
    chosen_rows = []
    for j in range(_E):
        sj = sm[j:j + 1, :]
        rank = jnp.sum((sm > sj).astype(jnp.int32), axis=0, keepdims=True)
        if j > 0:
            rank = rank + jnp.sum((sm[:j, :] == sj).astype(jnp.int32),
                                  axis=0, keepdims=True)
        is0 = rank == 0
        is1 = rank == 1
        e0 = jnp.where(is0, j, e0)
        p0 = jnp.where(is0, sj, p0)
        e1 = jnp.where(is1, j, e1)
        p1 = jnp.where(is1, sj, p1)
        chosen_rows.append((rank < 2).astype(jnp.float32))
    chosen = jnp.concatenate(chosen_rows, axis=0)

    row = lax.broadcasted_iota(jnp.int32, (_NW, _N), 1)
    wid = lax.broadcasted_iota(jnp.int32, (_NW, _N), 0)
    seg = (row // _TPW == wid).astype(jnp.float32)
    ws = lax.dot_general(seg, chosen, (((1,), (1,)), ((), ())),
                         preferred_element_type=jnp.float32)
    a = lax.broadcasted_iota(jnp.int32, (_NW, _NW), 0)
    b = lax.broadcasted_iota(jnp.int32, (_NW, _NW), 1)
    ltri = (b < a).astype(jnp.float32)
    pre = lax.dot_general(ltri, ws, (((1,), (0,)), ((), ())),
                          preferred_element_type=jnp.float32)
    total = jnp.sum(ws, axis=0, keepdims=True)
    blk = (total.astype(jnp.int32) + (_BLK - 1)) // _BLK
    ea = lax.broadcasted_iota(jnp.int32, (_E, _E), 0)
    eb = lax.broadcasted_iota(jnp.int32, (_E, _E), 1)
    ltri_e = (ea < eb).astype(jnp.float32)
    base_blk = lax.dot_general(blk.astype(jnp.float32), ltri_e,
                               (((1,), (0,)), ((), ())),
                               preferred_element_type=jnp.float32)
    base_blk = base_blk.astype(jnp.int32)
    off = base_blk * _BLK

    run0 = (off.astype(jnp.float32) + pre).astype(jnp.int32)
    run_ref[:, 0:_E] = run0
    run_ref[:, _E:_L] = jnp.zeros((_NW, _L - _E), jnp.int32)

    biota = lax.broadcasted_iota(jnp.int32, (1, _BEXP_PAD), 1)
    bexp = jnp.zeros((1, _BEXP_PAD), jnp.int32)
    for e in range(_E):
        be = base_blk[:, e:e + 1]
        ne = blk[:, e:e + 1]
        mask = (biota >= be) & (biota < be + ne)
        bexp = jnp.where(mask, e, bexp)
    bexp_ref[...] = bexp

    e0_ref[...] = e0
    e1_ref[...] = e1
    p0_ref[...] = p0
    p1_ref[...] = p1


def _router(x2, Wr):
    return pl.pallas_call(
        _router_kernel,
        out_shape=(
            jax.ShapeDtypeStruct((1, _N), jnp.int32),
            jax.ShapeDtypeStruct((1, _N), jnp.int32),
            jax.ShapeDtypeStruct((1, _N), jnp.float32),
            jax.ShapeDtypeStruct((1, _N), jnp.float32),
            jax.ShapeDtypeStruct((_NW, _L), jnp.int32),
            jax.ShapeDtypeStruct((1, _BEXP_PAD), jnp.int32),
        ),
    )(x2, Wr)


def _dispatch_body(x_hbm, e0_hbm, e1_hbm, run_hbm, xg_hbm, s0_hbm, s1_hbm,
                   e0_v, e1_v, s0_v, s1_v, run_v, xrows):
    w = lax.axis_index("s") * _NC_CORES + lax.axis_index("c")
    base = w * _TPW
    pltpu.sync_copy(e0_hbm.at[pl.ds(base, _TPW)], e0_v)
    pltpu.sync_copy(e1_hbm.at[pl.ds(base, _TPW)], e1_v)
    pltpu.sync_copy(run_hbm.at[w], run_v)
    run = run_v[...]
    ivec = lax.iota(jnp.int32, _L)
    for ci in range(_NCHUNK):
        tok0 = base + ci * _L
        pltpu.sync_copy(x_hbm.at[pl.ds(tok0, _L)], xrows)
        for which in range(2):
            ev = e0_v if which == 0 else e1_v
            c = ev[pl.ds(ci * _L, _L)]
            slot = jnp.zeros((_L,), jnp.int32)
            for e in range(_E):
                msk = c == e
                pref = plsc.cumsum(msk.astype(jnp.int32))
                base_e = jnp.sum(jnp.where(ivec == e, run, 0))
                slot = jnp.where(msk, base_e + pref - 1, slot)
                cnt = plsc.all_reduce_population_count(msk)
                run = jnp.where(ivec == e, run + cnt, run)
            pltpu.sync_copy(xrows, xg_hbm.at[slot])
            if which == 0:
                s0_v[pl.ds(ci * _L, _L)] = slot
            else:
                s1_v[pl.ds(ci * _L, _L)] = slot
    pltpu.sync_copy(s0_v, s0_hbm.at[pl.ds(base, _TPW)])
    pltpu.sync_copy(s1_v, s1_hbm.at[pl.ds(base, _TPW)])


_NC_CORES = 2


@functools.cache
def _dispatch_call():
    mesh = plsc.VectorSubcoreMesh(core_axis_name="c", subcore_axis_name="s")
    return functools.partial(
        pl.kernel,
        out_type=(
            jax.ShapeDtypeStruct((_S, _D), jnp.float32),
            jax.ShapeDtypeStruct((_N,), jnp.int32),
            jax.ShapeDtypeStruct((_N,), jnp.int32),
        ),
        mesh=mesh,
        compiler_params=pltpu.CompilerParams(needs_layout_passes=False),
        scratch_types=[
            pltpu.VMEM((_TPW,), jnp.int32),
            pltpu.VMEM((_TPW,), jnp.int32),
            pltpu.VMEM((_TPW,), jnp.int32),
            pltpu.VMEM((_TPW,), jnp.int32),
            pltpu.VMEM((_L,), jnp.int32),
            pltpu.VMEM((_L, _D), jnp.float32),
        ],
    )(_dispatch_body)


def _wcast_kernel(w1_ref, w2_ref, w1b_ref, w2b_ref):
    w1b_ref[...] = w1_ref[...].astype(jnp.bfloat16)
    w2b_ref[...] = w2_ref[...].astype(jnp.bfloat16)


def _wcast(W1, W2):
    return pl.pallas_call(
        _wcast_kernel,
        grid=(_E,),
        in_specs=[
            pl.BlockSpec((1, _F, _D), lambda e: (e, 0, 0)),
            pl.BlockSpec((1, _D, _F), lambda e: (e, 0, 0)),
        ],
        out_specs=(
            pl.BlockSpec((1, _F, _D), lambda e: (e, 0, 0)),
            pl.BlockSpec((1, _D, _F), lambda e: (e, 0, 0)),
        ),
        out_shape=(
            jax.ShapeDtypeStruct((_E, _F, _D), jnp.bfloat16),
            jax.ShapeDtypeStruct((_E, _D, _F), jnp.bfloat16),
        ),
        compiler_params=pltpu.CompilerParams(
            dimension_semantics=("arbitrary",)),
    )(W1, W2)


def _ffn_kernel(bexp_ref, xg_ref, w1_ref, w2_ref, y_ref):
    h = lax.dot_general(xg_ref[...], w1_ref[0], (((1,), (1,)), ((), ())),
                        preferred_element_type=jnp.float32)
    h = _gelu_tanh(h)
    y = lax.dot_general(h, w2_ref[0], (((1,), (1,)), ((), ())),
                        preferred_element_type=jnp.float32)
    y_ref[...] = y


def _ffn(bexp, xg, W1b, W2b):
    grid_spec = pltpu.PrefetchScalarGridSpec(
        num_scalar_prefetch=1,
        grid=(_NBLK,),
        in_specs=[
            pl.BlockSpec((_BLK, _D), lambda b, be: (b, 0)),
            pl.BlockSpec((1, _F, _D), lambda b, be: (be[b], 0, 0)),
            pl.BlockSpec((1, _D, _F), lambda b, be: (be[b], 0, 0)),
        ],
        out_specs=pl.BlockSpec((_BLK, _D), lambda b, be: (b, 0)),
    )
    return pl.pallas_call(
        _ffn_kernel,
        grid_spec=grid_spec,
        out_shape=jax.ShapeDtypeStruct((_S, _D), jnp.float32),
        compiler_params=pltpu.CompilerParams(
            dimension_semantics=("arbitrary",)),
    )(bexp, xg, W1b, W2b)


def _combine_body(y_hbm, s0_hbm, s1_hbm, p0_hbm, p1_hbm, out_hbm,
                  s0a_v, s1a_v, p0a_v, p1a_v, y0_v, y1_v, out_v,
                  sem_idx, sem_g0, sem_g1, sem_out):
    w = lax.axis_index("s") * _NC_CORES + lax.axis_index("c")
    base = w * _TPW
    d0 = pltpu.async_copy(s0_hbm.at[pl.ds(base, _TPW)], s0a_v, sem_idx)
    d1 = pltpu.async_copy(s1_hbm.at[pl.ds(base, _TPW)], s1a_v, sem_idx)
    d2 = pltpu.async_copy(p0_hbm.at[pl.ds(base, _TPW)], p0a_v, sem_idx)
    d3 = pltpu.async_copy(p1_hbm.at[pl.ds(base, _TPW)], p1a_v, sem_idx)
    d0.wait(); d1.wait(); d2.wait(); d3.wait()

    ivec = lax.iota(jnp.int32, _L)
    zf = jnp.zeros((_L,), jnp.float32)

    def issue_gathers(ci, buf):
        idx0 = s0a_v[pl.ds(ci * _L, _L)]
        idx1 = s1a_v[pl.ds(ci * _L, _L)]
        g0 = pltpu.async_copy(y_hbm.at[idx0], y0_v.at[buf], sem_g0)
        g1 = pltpu.async_copy(y_hbm.at[idx1], y1_v.at[buf], sem_g1)
        return g0, g1

    pend = issue_gathers(0, 0)
    wr_pend = None
    for ci in range(_NCHUNK):
        buf = ci % 2
        pend[0].wait()
        pend[1].wait()
        if ci + 1 < _NCHUNK:
            pend = issue_gathers(ci + 1, (ci + 1) % 2)
        p0vec = p0a_v[pl.ds(ci * _L, _L)]
        p1vec = p1a_v[pl.ds(ci * _L, _L)]

        def tbody(t, _):
            p0b = jnp.sum(jnp.where(ivec == t, p0vec, zf))
            p1b = jnp.sum(jnp.where(ivec == t, p1vec, zf))
            for j in range(_D // _L):
                y0c = y0_v[buf, t, pl.ds(j * _L, _L)]
                y1c = y1_v[buf, t, pl.ds(j * _L, _L)]
                out_v[buf, t, pl.ds(j * _L, _L)] = p0b * y0c + p1b * y1c
            return 0

        lax.fori_loop(0, _L, tbody, 0)
        if wr_pend is not None:
            wr_pend.wait()
        wr_pend = pltpu.async_copy(
            out_v.at[buf], out_hbm.at[pl.ds(base + ci * _L, _L)], sem_out)
    wr_pend.wait()


@functools.cache
def _combine_call():
    mesh = plsc.VectorSubcoreMesh(core_axis_name="c", subcore_axis_name="s")
    return functools.partial(
        pl.kernel,
        out_type=jax.ShapeDtypeStruct((_N, _D), jnp.float32),
        mesh=mesh,
        compiler_params=pltpu.CompilerParams(needs_layout_passes=False),
        scratch_types=[
            pltpu.VMEM((_TPW,), jnp.int32),
            pltpu.VMEM((_TPW,), jnp.int32),
            pltpu.VMEM((_TPW,), jnp.float32),
            pltpu.VMEM((_TPW,), jnp.float32),
            pltpu.VMEM((2, _L, _D), jnp.float32),
            pltpu.VMEM((2, _L, _D), jnp.float32),
            pltpu.VMEM((2, _L, _D), jnp.float32),
            pltpu.SemaphoreType.DMA,
            pltpu.SemaphoreType.DMA,
            pltpu.SemaphoreType.DMA,
            pltpu.SemaphoreType.DMA,
        ],
    )(_combine_body)


@jax.jit
def kernel(x, Wr, W1, W2):
    Bb, Ll, Dd = x.shape
    x2 = x.reshape(_N, _D)
    e0, e1, p0, p1, run0, bexp = _router(x2, Wr)
    e0 = e0.reshape(_N)
    e1 = e1.reshape(_N)
    p0 = p0.reshape(_N)
    p1 = p1.reshape(_N)
    bexp = bexp.reshape(_BEXP_PAD)
    xg, s0, s1 = _dispatch_call()(x2, e0, e1, run0)
    y = _ffn(bexp, xg, W1, W2)
    out = _combine_call()(y, s0, s1, p0, p1)
    return out.reshape(Bb, Ll, Dd)


# speedup vs baseline: 3.1888x; 1.0377x over previous
import functools

import jax
import jax.numpy as jnp
from jax import lax
from jax.experimental import pallas as pl
from jax.experimental.pallas import tpu as pltpu
from jax.experimental.pallas import tpu_sc as plsc

_D = 1024
_E = 8
_F = 1024
_N = 2048
_L = 16
_NW = 32
_TPW = _N // _NW
_NCHUNK = _TPW // _L
_BLK = 512
_NBLK = 15
_S = _NBLK * _BLK
_BEXP_PAD = 48


def _gelu_tanh(h):
    c = 0.7978845608028654
    return 0.5 * h * (1.0 + jnp.tanh(c * (h + 0.044715 * h * h * h)))


def _router_kernel(x_ref, wr_ref, e0_ref, e1_ref, p0_ref, p1_ref,
                   run_ref, bexp_ref):
    xf = x_ref[...]
    logits = lax.dot_general(wr_ref[...], xf, (((1,), (1,)), ((), ())),
                             preferred_element_type=jnp.float32)
    m = jnp.max(logits, axis=0, keepdims=True)
    ex = jnp.exp(logits - m)
    sm = ex / jnp.sum(ex, axis=0, keepdims=True)

    e0 = jnp.zeros((1, _N), jnp.int32)
    e1 = jnp.zeros((1, _N), jnp.int32)
    p0 = jnp.zeros((1, _N), jnp.float32)
    p1 = jnp.zeros((1, _N), jnp.float32)
    chosen_rows = []
    for j in range(_E):
        sj = sm[j:j + 1, :]
        rank = jnp.sum((sm > sj).astype(jnp.int32), axis=0, keepdims=True)
        if j > 0:
            rank = rank + jnp.sum((sm[:j, :] == sj).astype(jnp.int32),
                                  axis=0, keepdims=True)
        is0 = rank == 0
        is1 = rank == 1
        e0 = jnp.where(is0, j, e0)
        p0 = jnp.where(is0, sj, p0)
        e1 = jnp.where(is1, j, e1)
        p1 = jnp.where(is1, sj, p1)
        chosen_rows.append((rank < 2).astype(jnp.float32))
    chosen = jnp.concatenate(chosen_rows, axis=0)

    row = lax.broadcasted_iota(jnp.int32, (_NW, _N), 1)
    wid = lax.broadcasted_iota(jnp.int32, (_NW, _N), 0)
    seg = (row // _TPW == wid).astype(jnp.float32)
    ws = lax.dot_general(seg, chosen, (((1,), (1,)), ((), ())),
                         preferred_element_type=jnp.float32)
    a = lax.broadcasted_iota(jnp.int32, (_NW, _NW), 0)
    b = lax.broadcasted_iota(jnp.int32, (_NW, _NW), 1)
    ltri = (b < a).astype(jnp.float32)
    pre = lax.dot_general(ltri, ws, (((1,), (0,)), ((), ())),
                          preferred_element_type=jnp.float32)
    total = jnp.sum(ws, axis=0, keepdims=True)
    blk = (total.astype(jnp.int32) + (_BLK - 1)) // _BLK
    ea = lax.broadcasted_iota(jnp.int32, (_E, _E), 0)
    eb = lax.broadcasted_iota(jnp.int32, (_E, _E), 1)
    ltri_e = (ea < eb).astype(jnp.float32)
    base_blk = lax.dot_general(blk.astype(jnp.float32), ltri_e,
                               (((1,), (0,)), ((), ())),
                               preferred_element_type=jnp.float32)
    base_blk = base_blk.astype(jnp.int32)
    off = base_blk * _BLK

    run0 = (off.astype(jnp.float32) + pre).astype(jnp.int32)
    run_ref[:, 0:_E] = run0
    run_ref[:, _E:_L] = jnp.zeros((_NW, _L - _E), jnp.int32)

    biota = lax.broadcasted_iota(jnp.int32, (1, _BEXP_PAD), 1)
    bexp = jnp.zeros((1, _BEXP_PAD), jnp.int32)
    for e in range(_E):
        be = base_blk[:, e:e + 1]
        ne = blk[:, e:e + 1]
        mask = (biota >= be) & (biota < be + ne)
        bexp = jnp.where(mask, e, bexp)
    bexp_ref[...] = bexp

    e0_ref[...] = e0
    e1_ref[...] = e1
    p0_ref[...] = p0
    p1_ref[...] = p1


def _router(x2, Wr):
    return pl.pallas_call(
        _router_kernel,
        out_shape=(
            jax.ShapeDtypeStruct((1, _N), jnp.int32),
            jax.ShapeDtypeStruct((1, _N), jnp.int32),
            jax.ShapeDtypeStruct((1, _N), jnp.float32),
            jax.ShapeDtypeStruct((1, _N), jnp.float32),
            jax.ShapeDtypeStruct((_NW, _L), jnp.int32),
            jax.ShapeDtypeStruct((1, _BEXP_PAD), jnp.int32),
        ),
    )(x2, Wr)


def _dispatch_body(x_hbm, e0_hbm, e1_hbm, run_hbm, xg_hbm, s0_hbm, s1_hbm,
                   e0_v, e1_v, s0_v, s1_v, run_v, xrows, sem_x, sem_s):
    w = lax.axis_index("s") * _NC_CORES + lax.axis_index("c")
    base = w * _TPW
    dx = pltpu.async_copy(x_hbm.at[pl.ds(base, _TPW)], xrows, sem_x)
    pltpu.sync_copy(e0_hbm.at[pl.ds(base, _TPW)], e0_v)
    pltpu.sync_copy(e1_hbm.at[pl.ds(base, _TPW)], e1_v)
    pltpu.sync_copy(run_hbm.at[w], run_v)
    run = run_v[...]
    ivec = lax.iota(jnp.int32, _L)
    for ci in range(_NCHUNK):
        for which in range(2):
            ev = e0_v if which == 0 else e1_v
            c = ev[pl.ds(ci * _L, _L)]
            slot = jnp.zeros((_L,), jnp.int32)
            for e in range(_E):
                msk = c == e
                pref = plsc.cumsum(msk.astype(jnp.int32))
                base_e = jnp.sum(jnp.where(ivec == e, run, 0))
                slot = jnp.where(msk, base_e + pref - 1, slot)
                cnt = plsc.all_reduce_population_count(msk)
                run = jnp.where(ivec == e, run + cnt, run)
            if which == 0:
                s0_v[pl.ds(ci * _L, _L)] = slot
            else:
                s1_v[pl.ds(ci * _L, _L)] = slot
    dx.wait()
    d0 = pltpu.async_copy(xrows, xg_hbm.at[s0_v], sem_s)
    d1 = pltpu.async_copy(xrows, xg_hbm.at[s1_v], sem_s)
    pltpu.sync_copy(s0_v, s0_hbm.at[pl.ds(base, _TPW)])
    pltpu.sync_copy(s1_v, s1_hbm.at[pl.ds(base, _TPW)])
    d0.wait()
    d1.wait()


_NC_CORES = 2


@functools.cache
def _dispatch_call():
    mesh = plsc.VectorSubcoreMesh(core_axis_name="c", subcore_axis_name="s")
    return functools.partial(
        pl.kernel,
        out_type=(
            jax.ShapeDtypeStruct((_S, _D), jnp.float32),
            jax.ShapeDtypeStruct((_N,), jnp.int32),
            jax.ShapeDtypeStruct((_N,), jnp.int32),
        ),
        mesh=mesh,
        compiler_params=pltpu.CompilerParams(needs_layout_passes=False),
        scratch_types=[
            pltpu.VMEM((_TPW,), jnp.int32),
            pltpu.VMEM((_TPW,), jnp.int32),
            pltpu.VMEM((_TPW,), jnp.int32),
            pltpu.VMEM((_TPW,), jnp.int32),
            pltpu.VMEM((_L,), jnp.int32),
            pltpu.VMEM((_TPW, _D), jnp.float32),
            pltpu.SemaphoreType.DMA,
            pltpu.SemaphoreType.DMA,
        ],
    )(_dispatch_body)


def _wcast_kernel(w1_ref, w2_ref, w1b_ref, w2b_ref):
    w1b_ref[...] = w1_ref[...].astype(jnp.bfloat16)
    w2b_ref[...] = w2_ref[...].astype(jnp.bfloat16)


def _wcast(W1, W2):
    return pl.pallas_call(
        _wcast_kernel,
        grid=(_E,),
        in_specs=[
            pl.BlockSpec((1, _F, _D), lambda e: (e, 0, 0)),
            pl.BlockSpec((1, _D, _F), lambda e: (e, 0, 0)),
        ],
        out_specs=(
            pl.BlockSpec((1, _F, _D), lambda e: (e, 0, 0)),
            pl.BlockSpec((1, _D, _F), lambda e: (e, 0, 0)),
        ),
        out_shape=(
            jax.ShapeDtypeStruct((_E, _F, _D), jnp.bfloat16),
            jax.ShapeDtypeStruct((_E, _D, _F), jnp.bfloat16),
        ),
        compiler_params=pltpu.CompilerParams(
            dimension_semantics=("arbitrary",)),
    )(W1, W2)


def _ffn_kernel(bexp_ref, xg_ref, w1_ref, w2_ref, y_ref):
    h = lax.dot_general(xg_ref[...], w1_ref[0], (((1,), (1,)), ((), ())),
                        preferred_element_type=jnp.float32)
    h = _gelu_tanh(h.astype(jnp.bfloat16))
    y = lax.dot_general(h, w2_ref[0], (((1,), (1,)), ((), ())),
                        preferred_element_type=jnp.float32)
    y_ref[...] = y


def _ffn(bexp, xg, W1b, W2b):
    grid_spec = pltpu.PrefetchScalarGridSpec(
        num_scalar_prefetch=1,
        grid=(_NBLK,),
        in_specs=[
            pl.BlockSpec((_BLK, _D), lambda b, be: (b, 0)),
            pl.BlockSpec((1, _F, _D), lambda b, be: (be[b], 0, 0)),
            pl.BlockSpec((1, _D, _F), lambda b, be: (be[b], 0, 0)),
        ],
        out_specs=pl.BlockSpec((_BLK, _D), lambda b, be: (b, 0)),
    )
    return pl.pallas_call(
        _ffn_kernel,
        grid_spec=grid_spec,
        out_shape=jax.ShapeDtypeStruct((_S, _D), jnp.float32),
        compiler_params=pltpu.CompilerParams(
            dimension_semantics=("arbitrary",)),
    )(bexp, xg, W1b, W2b)


def _combine_body(y_hbm, s0_hbm, s1_hbm, p0_hbm, p1_hbm, out_hbm,
                  s0a_v, s1a_v, p0a_v, p1a_v, y0_v, y1_v, out_v,
                  sem_idx, sem_g0, sem_g1, sem_out):
    w = lax.axis_index("s") * _NC_CORES + lax.axis_index("c")
    base = w * _TPW
    d0 = pltpu.async_copy(s0_hbm.at[pl.ds(base, _TPW)], s0a_v, sem_idx)
    d1 = pltpu.async_copy(s1_hbm.at[pl.ds(base, _TPW)], s1a_v, sem_idx)
    d2 = pltpu.async_copy(p0_hbm.at[pl.ds(base, _TPW)], p0a_v, sem_idx)
    d3 = pltpu.async_copy(p1_hbm.at[pl.ds(base, _TPW)], p1a_v, sem_idx)
    d0.wait(); d1.wait(); d2.wait(); d3.wait()

    ivec = lax.iota(jnp.int32, _L)
    zf = jnp.zeros((_L,), jnp.float32)

    def issue_gathers(ci, buf):
        idx0 = s0a_v[pl.ds(ci * _L, _L)]
        idx1 = s1a_v[pl.ds(ci * _L, _L)]
        g0 = pltpu.async_copy(y_hbm.at[idx0], y0_v.at[buf], sem_g0)
        g1 = pltpu.async_copy(y_hbm.at[idx1], y1_v.at[buf], sem_g1)
        return g0, g1

    pend = issue_gathers(0, 0)
    wr_pend = None
    for ci in range(_NCHUNK):
        buf = ci % 2
        pend[0].wait()
        pend[1].wait()
        if ci + 1 < _NCHUNK:
            pend = issue_gathers(ci + 1, (ci + 1) % 2)
        p0vec = p0a_v[pl.ds(ci * _L, _L)]
        p1vec = p1a_v[pl.ds(ci * _L, _L)]

        def tbody(t, _):
            p0b = jnp.sum(jnp.where(ivec == t, p0vec, zf))
            p1b = jnp.sum(jnp.where(ivec == t, p1vec, zf))
            for j in range(_D // _L):
                y0c = y0_v[buf, t, pl.ds(j * _L, _L)]
                y1c = y1_v[buf, t, pl.ds(j * _L, _L)]
                out_v[buf, t, pl.ds(j * _L, _L)] = p0b * y0c + p1b * y1c
            return 0

        lax.fori_loop(0, _L, tbody, 0)
        if wr_pend is not None:
            wr_pend.wait()
        wr_pend = pltpu.async_copy(
            out_v.at[buf], out_hbm.at[pl.ds(base + ci * _L, _L)], sem_out)
    wr_pend.wait()


@functools.cache
def _combine_call():
    mesh = plsc.VectorSubcoreMesh(core_axis_name="c", subcore_axis_name="s")
    return functools.partial(
        pl.kernel,
        out_type=jax.ShapeDtypeStruct((_N, _D), jnp.float32),
        mesh=mesh,
        compiler_params=pltpu.CompilerParams(needs_layout_passes=False),
        scratch_types=[
            pltpu.VMEM((_TPW,), jnp.int32),
            pltpu.VMEM((_TPW,), jnp.int32),
            pltpu.VMEM((_TPW,), jnp.float32),
            pltpu.VMEM((_TPW,), jnp.float32),
            pltpu.VMEM((2, _L, _D), jnp.float32),
            pltpu.VMEM((2, _L, _D), jnp.float32),
            pltpu.VMEM((2, _L, _D), jnp.float32),
            pltpu.SemaphoreType.DMA,
            pltpu.SemaphoreType.DMA,
            pltpu.SemaphoreType.DMA,
            pltpu.SemaphoreType.DMA,
        ],
    )(_combine_body)


@jax.jit
def kernel(x, Wr, W1, W2):
    Bb, Ll, Dd = x.shape
    x2 = x.reshape(_N, _D)
    e0, e1, p0, p1, run0, bexp = _router(x2, Wr)
    e0 = e0.reshape(_N)
    e1 = e1.reshape(_N)
    p0 = p0.reshape(_N)
    p1 = p1.reshape(_N)
    bexp = bexp.reshape(_BEXP_PAD)
    xg, s0, s1 = _dispatch_call()(x2, e0, e1, run0)
    y = _ffn(bexp, xg, W1, W2)
    out = _combine_call()(y, s0, s1, p0, p1)
    return out.reshape(Bb, Ll, Dd)


# speedup vs baseline: 3.1900x; 1.0004x over previous
import functools

import jax
import jax.numpy as jnp
from jax import lax
from jax.experimental import pallas as pl
from jax.experimental.pallas import tpu as pltpu
from jax.experimental.pallas import tpu_sc as plsc

_D = 1024
_E = 8
_F = 1024
_N = 2048
_L = 16
_NW = 32
_TPW = _N // _NW
_NCHUNK = _TPW // _L
_BLK = 512
_NBLK = 15
_S = _NBLK * _BLK
_BEXP_PAD = 48


def _gelu_tanh(h):
    c = 0.7978845608028654
    return 0.5 * h * (1.0 + jnp.tanh(c * (h + 0.044715 * h * h * h)))


def _router_kernel(x_ref, wr_ref, e0_ref, e1_ref, p0_ref, p1_ref,
                   run_ref, bexp_ref):
    xf = x_ref[...]
    logits = lax.dot_general(wr_ref[...], xf, (((1,), (1,)), ((), ())),
                             preferred_element_type=jnp.float32)
    m = jnp.max(logits, axis=0, keepdims=True)
    ex = jnp.exp(logits - m)
    sm = ex / jnp.sum(ex, axis=0, keepdims=True)

    e0 = jnp.zeros((1, _N), jnp.int32)
    e1 = jnp.zeros((1, _N), jnp.int32)
    p0 = jnp.zeros((1, _N), jnp.float32)
    p1 = jnp.zeros((1, _N), jnp.float32)
    chosen_rows = []
    for j in range(_E):
        sj = sm[j:j + 1, :]
        rank = jnp.sum((sm > sj).astype(jnp.int32), axis=0, keepdims=True)
        if j > 0:
            rank = rank + jnp.sum((sm[:j, :] == sj).astype(jnp.int32),
                                  axis=0, keepdims=True)
        is0 = rank == 0
        is1 = rank == 1
        e0 = jnp.where(is0, j, e0)
        p0 = jnp.where(is0, sj, p0)
        e1 = jnp.where(is1, j, e1)
        p1 = jnp.where(is1, sj, p1)
        chosen_rows.append((rank < 2).astype(jnp.float32))
    chosen = jnp.concatenate(chosen_rows, axis=0)

    row = lax.broadcasted_iota(jnp.int32, (_NW, _N), 1)
    wid = lax.broadcasted_iota(jnp.int32, (_NW, _N), 0)
    seg = (row // _TPW == wid).astype(jnp.float32)
    ws = lax.dot_general(seg, chosen, (((1,), (1,)), ((), ())),
                         preferred_element_type=jnp.float32)
    a = lax.broadcasted_iota(jnp.int32, (_NW, _NW), 0)
    b = lax.broadcasted_iota(jnp.int32, (_NW, _NW), 1)
    ltri = (b < a).astype(jnp.float32)
    pre = lax.dot_general(ltri, ws, (((1,), (0,)), ((), ())),
                          preferred_element_type=jnp.float32)
    total = jnp.sum(ws, axis=0, keepdims=True)
    blk = (total.astype(jnp.int32) + (_BLK - 1)) // _BLK
    ea = lax.broadcasted_iota(jnp.int32, (_E, _E), 0)
    eb = lax.broadcasted_iota(jnp.int32, (_E, _E), 1)
    ltri_e = (ea < eb).astype(jnp.float32)
    base_blk = lax.dot_general(blk.astype(jnp.float32), ltri_e,
                               (((1,), (0,)), ((), ())),
                               preferred_element_type=jnp.float32)
    base_blk = base_blk.astype(jnp.int32)
    off = base_blk * _BLK

    run0 = (off.astype(jnp.float32) + pre).astype(jnp.int32)
    run_ref[:, 0:_E] = run0
    run_ref[:, _E:_L] = jnp.zeros((_NW, _L - _E), jnp.int32)

    biota = lax.broadcasted_iota(jnp.int32, (1, _BEXP_PAD), 1)
    bexp = jnp.zeros((1, _BEXP_PAD), jnp.int32)
    for e in range(_E):
        be = base_blk[:, e:e + 1]
        ne = blk[:, e:e + 1]
        mask = (biota >= be) & (biota < be + ne)
        bexp = jnp.where(mask, e, bexp)
    bexp_ref[...] = bexp

    e0_ref[...] = e0
    e1_ref[...] = e1
    p0_ref[...] = p0
    p1_ref[...] = p1


def _router(x2, Wr):
    return pl.pallas_call(
        _router_kernel,
        out_shape=(
            jax.ShapeDtypeStruct((1, _N), jnp.int32),
            jax.ShapeDtypeStruct((1, _N), jnp.int32),
            jax.ShapeDtypeStruct((1, _N), jnp.float32),
            jax.ShapeDtypeStruct((1, _N), jnp.float32),
            jax.ShapeDtypeStruct((_NW, _L), jnp.int32),
            jax.ShapeDtypeStruct((1, _BEXP_PAD), jnp.int32),
        ),
    )(x2, Wr)


def _dispatch_body(x_hbm, e0_hbm, e1_hbm, run_hbm, xg_hbm, s0_hbm, s1_hbm,
                   e0_v, e1_v, s0_v, s1_v, run_v, xrows, sem_x, sem_s):
    w = lax.axis_index("s") * _NC_CORES + lax.axis_index("c")
    base = w * _TPW
    dx = pltpu.async_copy(x_hbm.at[pl.ds(base, _TPW)], xrows, sem_x)
    pltpu.sync_copy(e0_hbm.at[pl.ds(base, _TPW)], e0_v)
    pltpu.sync_copy(e1_hbm.at[pl.ds(base, _TPW)], e1_v)
    pltpu.sync_copy(run_hbm.at[w], run_v)
    run = run_v[...]
    ivec = lax.iota(jnp.int32, _L)
    for ci in range(_NCHUNK):
        for which in range(2):
            ev = e0_v if which == 0 else e1_v
            c = ev[pl.ds(ci * _L, _L)]
            slot = jnp.zeros((_L,), jnp.int32)
            for e in range(_E):
                msk = c == e
                pref = plsc.cumsum(msk.astype(jnp.int32))
                base_e = jnp.sum(jnp.where(ivec == e, run, 0))
                slot = jnp.where(msk, base_e + pref - 1, slot)
                cnt = plsc.all_reduce_population_count(msk)
                run = jnp.where(ivec == e, run + cnt, run)
            if which == 0:
                s0_v[pl.ds(ci * _L, _L)] = slot
            else:
                s1_v[pl.ds(ci * _L, _L)] = slot
    dx.wait()
    d0 = pltpu.async_copy(xrows, xg_hbm.at[s0_v], sem_s)
    d1 = pltpu.async_copy(xrows, xg_hbm.at[s1_v], sem_s)
    pltpu.sync_copy(s0_v, s0_hbm.at[pl.ds(base, _TPW)])
    pltpu.sync_copy(s1_v, s1_hbm.at[pl.ds(base, _TPW)])
    d0.wait()
    d1.wait()


_NC_CORES = 2


@functools.cache
def _dispatch_call():
    mesh = plsc.VectorSubcoreMesh(core_axis_name="c", subcore_axis_name="s")
    return functools.partial(
        pl.kernel,
        out_type=(
            jax.ShapeDtypeStruct((_S, _D), jnp.float32),
            jax.ShapeDtypeStruct((_N,), jnp.int32),
            jax.ShapeDtypeStruct((_N,), jnp.int32),
        ),
        mesh=mesh,
        compiler_params=pltpu.CompilerParams(needs_layout_passes=False),
        scratch_types=[
            pltpu.VMEM((_TPW,), jnp.int32),
            pltpu.VMEM((_TPW,), jnp.int32),
            pltpu.VMEM((_TPW,), jnp.int32),
            pltpu.VMEM((_TPW,), jnp.int32),
            pltpu.VMEM((_L,), jnp.int32),
            pltpu.VMEM((_TPW, _D), jnp.float32),
            pltpu.SemaphoreType.DMA,
            pltpu.SemaphoreType.DMA,
        ],
    )(_dispatch_body)


def _ffn_kernel(bexp_ref, xg_ref, w1_ref, w2_ref, y_ref):
    h = lax.dot_general(xg_ref[...], w1_ref[0], (((1,), (1,)), ((), ())),
                        preferred_element_type=jnp.float32)
    h = _gelu_tanh(h.astype(jnp.bfloat16))
    y = lax.dot_general(h, w2_ref[0], (((1,), (1,)), ((), ())),
                        preferred_element_type=jnp.float32)
    y_ref[...] = y


def _ffn(bexp, xg, W1b, W2b):
    grid_spec = pltpu.PrefetchScalarGridSpec(
        num_scalar_prefetch=1,
        grid=(_NBLK,),
        in_specs=[
            pl.BlockSpec((_BLK, _D), lambda b, be: (b, 0)),
            pl.BlockSpec((1, _F, _D), lambda b, be: (be[b], 0, 0)),
            pl.BlockSpec((1, _D, _F), lambda b, be: (be[b], 0, 0)),
        ],
        out_specs=pl.BlockSpec((_BLK, _D), lambda b, be: (b, 0)),
    )
    return pl.pallas_call(
        _ffn_kernel,
        grid_spec=grid_spec,
        out_shape=jax.ShapeDtypeStruct((_S, _D), jnp.float32),
        compiler_params=pltpu.CompilerParams(
            dimension_semantics=("arbitrary",)),
    )(bexp, xg, W1b, W2b)


def _combine_body(y_hbm, s0_hbm, s1_hbm, p0_hbm, p1_hbm, out_hbm,
                  s0a_v, s1a_v, p0a_v, p1a_v, y0_v, y1_v, out_v,
                  sem_idx, sem_g0, sem_g1, sem_out):
    w = lax.axis_index("s") * _NC_CORES + lax.axis_index("c")
    base = w * _TPW
    d0 = pltpu.async_copy(s0_hbm.at[pl.ds(base, _TPW)], s0a_v, sem_idx)
    d1 = pltpu.async_copy(s1_hbm.at[pl.ds(base, _TPW)], s1a_v, sem_idx)
    d2 = pltpu.async_copy(p0_hbm.at[pl.ds(base, _TPW)], p0a_v, sem_idx)
    d3 = pltpu.async_copy(p1_hbm.at[pl.ds(base, _TPW)], p1a_v, sem_idx)
    d0.wait(); d1.wait(); d2.wait(); d3.wait()

    ivec = lax.iota(jnp.int32, _L)
    zf = jnp.zeros((_L,), jnp.float32)

    def issue_gathers(ci, buf):
        idx0 = s0a_v[pl.ds(ci * _L, _L)]
        idx1 = s1a_v[pl.ds(ci * _L, _L)]
        g0 = pltpu.async_copy(y_hbm.at[idx0], y0_v.at[buf], sem_g0)
        g1 = pltpu.async_copy(y_hbm.at[idx1], y1_v.at[buf], sem_g1)
        return g0, g1

    pend = issue_gathers(0, 0)
    wr_pend = None
    for ci in range(_NCHUNK):
        buf = ci % 2
        pend[0].wait()
        pend[1].wait()
        if ci + 1 < _NCHUNK:
            pend = issue_gathers(ci + 1, (ci + 1) % 2)
        p0vec = p0a_v[pl.ds(ci * _L, _L)]
        p1vec = p1a_v[pl.ds(ci * _L, _L)]

        def tbody(t, _):
            p0b = jnp.sum(jnp.where(ivec == t, p0vec, zf))
            p1b = jnp.sum(jnp.where(ivec == t, p1vec, zf))
            for j in range(_D // _L):
                y0c = y0_v[buf, t, pl.ds(j * _L, _L)]
                y1c = y1_v[buf, t, pl.ds(j * _L, _L)]
                out_v[buf, t, pl.ds(j * _L, _L)] = p0b * y0c + p1b * y1c
            return 0

        lax.fori_loop(0, _L, tbody, 0)
        if wr_pend is not None:
            wr_pend.wait()
        wr_pend = pltpu.async_copy(
            out_v.at[buf], out_hbm.at[pl.ds(base + ci * _L, _L)], sem_out)
    wr_pend.wait()


@functools.cache
def _combine_call():
    mesh = plsc.VectorSubcoreMesh(core_axis_name="c", subcore_axis_name="s")
    return functools.partial(
        pl.kernel,
        out_type=jax.ShapeDtypeStruct((_N, _D), jnp.float32),
        mesh=mesh,
        compiler_params=pltpu.CompilerParams(needs_layout_passes=False),
        scratch_types=[
            pltpu.VMEM((_TPW,), jnp.int32),
            pltpu.VMEM((_TPW,), jnp.int32),
            pltpu.VMEM((_TPW,), jnp.float32),
            pltpu.VMEM((_TPW,), jnp.float32),
            pltpu.VMEM((2, _L, _D), jnp.float32),
            pltpu.VMEM((2, _L, _D), jnp.float32),
            pltpu.VMEM((2, _L, _D), jnp.float32),
            pltpu.SemaphoreType.DMA,
            pltpu.SemaphoreType.DMA,
            pltpu.SemaphoreType.DMA,
            pltpu.SemaphoreType.DMA,
        ],
    )(_combine_body)


@jax.jit
def kernel(x, Wr, W1, W2):
    Bb, Ll, Dd = x.shape
    x2 = x.reshape(_N, _D)
    e0, e1, p0, p1, run0, bexp = _router(x2, Wr)
    e0 = e0.reshape(_N)
    e1 = e1.reshape(_N)
    p0 = p0.reshape(_N)
    p1 = p1.reshape(_N)
    bexp = bexp.reshape(_BEXP_PAD)
    xg, s0, s1 = _dispatch_call()(x2, e0, e1, run0)
    y = _ffn(bexp, xg, W1, W2)
    out = _combine_call()(y, s0, s1, p0, p1)
    return out.reshape(Bb, Ll, Dd)
